# baseline (speedup 1.0000x reference)
import functools

import jax
import jax.numpy as jnp
from jax import lax
from jax.experimental import pallas as pl
from jax.experimental.pallas import tpu as pltpu
from jax.experimental.pallas import tpu_sc as plsc

_THRESH = (0.5333333611488342, 0.6000000238418579, 0.6666666865348816,
           0.7333333492279053, 0.8000000715255737, 0.8666667342185974,
           0.9333333969116211)

_ROWS = 8192
_COLS = 2048
_BLOCK_ROWS = 256
_GRID = _ROWS // _BLOCK_ROWS


def _ece_body(x_ref, m_ref, t_ref, out_ref):
    @pl.when(pl.program_id(0) == 0)
    def _init():
        for k in range(24):
            out_ref[k] = 0.0

    x = x_ref[...]
    mf = m_ref[...].astype(jnp.float32)
    t = t_ref[...]
    conf = 0.5 * jnp.tanh(0.5 * jnp.abs(x)) + 0.5
    acc = jnp.where(x > 0, t, 1.0 - t) * mf
    confm = conf * mf
    out_ref[0] += jnp.sum(mf)
    out_ref[1] += jnp.sum(confm)
    out_ref[2] += jnp.sum(acc)
    for k, th in enumerate(_THRESH):
        g = conf > th
        base = 3 * (k + 1)
        out_ref[base + 0] += jnp.sum(jnp.where(g, mf, 0.0))
        out_ref[base + 1] += jnp.sum(jnp.where(g, confm, 0.0))
        out_ref[base + 2] += jnp.sum(jnp.where(g, acc, 0.0))


def _partial_sums(logits, mask, targets, interpret=False):
    blk = pl.BlockSpec((_BLOCK_ROWS, _COLS), lambda i: (i, 0))
    return pl.pallas_call(
        _ece_body,
        grid=(_GRID,),
        in_specs=[blk, blk, blk],
        out_specs=pl.BlockSpec(memory_space=pltpu.SMEM),
        out_shape=jax.ShapeDtypeStruct((24,), jnp.float32),
        interpret=interpret,
    )(logits, mask, targets)


_NW = 32
_ELEMS = _ROWS * _COLS
_PER_TILE = _ELEMS // _NW
_CHUNK = 16384
_NPIECE = _PER_TILE // _CHUNK
_NVEC = _CHUNK // 16
_UNROLL = 8


def _sc_hist_call(logits_flat, maskw, targets_flat):
    mesh = plsc.VectorSubcoreMesh(core_axis_name="c", subcore_axis_name="s")

    @functools.partial(
        pl.kernel,
        out_type=jax.ShapeDtypeStruct((_NW, 768), jnp.float32),
        mesh=mesh,
        compiler_params=pltpu.CompilerParams(needs_layout_passes=False),
        scratch_types=[
            pltpu.VMEM((2 * _CHUNK,), jnp.float32),
            pltpu.VMEM((2 * _CHUNK,), jnp.float32),
            pltpu.VMEM((2 * _CHUNK // 4,), jnp.int32),
        ] + [pltpu.VMEM((768,), jnp.float32) for _ in range(_UNROLL)] + [
            pltpu.SemaphoreType.DMA((2,)),
            pltpu.SemaphoreType.DMA((2,)),
            pltpu.SemaphoreType.DMA((2,)),
        ],
    )
    def sc_ece(x_hbm, mw_hbm, t_hbm, out_hbm, xbuf, tbuf, mbuf, *rest):
        hists = rest[:_UNROLL]
        xsem, tsem, msem = rest[_UNROLL:]
        wid = lax.axis_index("s") * 2 + lax.axis_index("c")
        base = wid * _PER_TILE

        zeros16 = jnp.zeros((16,), jnp.float32)
        for h in hists:
            for k in range(48):
                h[pl.ds(16 * k, 16)] = zeros16

        iota = lax.iota(jnp.int32, 16)
        lane_off = iota * 16
        word_sel = iota >> 2
        byte_shift = (iota & 3) * 8
        ones16 = jnp.ones((16,), jnp.float32)

        def slot_refs(p):
            slot = p & 1
            sbase = pl.multiple_of(slot * _CHUNK, _CHUNK)
            swbase = pl.multiple_of(slot * (_CHUNK // 4), _CHUNK // 4)
            return (xbuf.at[pl.ds(sbase, _CHUNK)],
                    tbuf.at[pl.ds(sbase, _CHUNK)],
                    mbuf.at[pl.ds(swbase, _CHUNK // 4)], slot)

        def start_piece(p):
            xb, tb, mb, slot = slot_refs(p)
            pbase = pl.multiple_of(base + p * _CHUNK, _CHUNK)
            wbase = pl.multiple_of(pbase // 4, _CHUNK // 4)
            pltpu.async_copy(x_hbm.at[pl.ds(pbase, _CHUNK)], xb, xsem.at[slot])
            pltpu.async_copy(t_hbm.at[pl.ds(pbase, _CHUNK)], tb, tsem.at[slot])
            pltpu.async_copy(mw_hbm.at[pl.ds(wbase, _CHUNK // 4)], mb,
                             msem.at[slot])

        def wait_piece(p):
            xb, tb, mb, slot = slot_refs(p)
            pltpu.make_async_copy(x_hbm.at[pl.ds(0, _CHUNK)], xb,
                                  xsem.at[slot]).wait()
            pltpu.make_async_copy(t_hbm.at[pl.ds(0, _CHUNK)], tb,
                                  tsem.at[slot]).wait()
            pltpu.make_async_copy(mw_hbm.at[pl.ds(0, _CHUNK // 4)], mb,
                                  msem.at[slot]).wait()

        start_piece(0)

        def piece_body(p, _):
            wait_piece(p)

            @pl.when(p + 1 < _NPIECE)
            def _prefetch():
                start_piece(p + 1)

            xb, tb, mb_ref, slot = slot_refs(p)

            def vec_body(v, _):
                for u in range(_UNROLL):
                    i = _UNROLL * v + u
                    off = pl.multiple_of(16 * i, 16)
                    x = xb[pl.ds(off, 16)]
                    t = tb[pl.ds(off, 16)]
                    w = plsc.load_gather(mb_ref, [4 * i + word_sel])
                    m = (w >> byte_shift) & 1
                    mb = m == 1
                    conf = 1.0 / (1.0 + jnp.exp(-jnp.abs(x)))
                    acc = jnp.where(x > 0.0, t, 1.0 - t)
                    b = (conf * 15.0).astype(jnp.int32)
                    idx = lane_off + b
                    h = hists[u]
                    plsc.addupdate_scatter(h, [idx], ones16, mask=mb)
                    plsc.addupdate_scatter(h, [idx + 256], conf, mask=mb)
                    plsc.addupdate_scatter(h, [idx + 512], acc, mask=mb)
                return 0

            lax.fori_loop(0, _NVEC // _UNROLL, vec_body, 0)
            return 0

        lax.fori_loop(0, _NPIECE, piece_body, 0)
        for k in range(48):
            sl = pl.ds(16 * k, 16)
            tot = hists[0][sl]
            for h in hists[1:]:
                tot = tot + h[sl]
            hists[0][sl] = tot
        pltpu.sync_copy(hists[0], out_hbm.at[wid])

    return sc_ece(logits_flat, maskw, targets_flat)


def _sc_kernel(logits, mask, targets):
    n = logits.size
    maskw = lax.bitcast_convert_type(
        mask.view(jnp.uint8).reshape(n // 4, 4), jnp.int32)
    part = _sc_hist_call(logits.reshape(n), maskw, targets.reshape(n))
    sums = part.reshape(_NW, 3, 16, 16).sum(axis=(0, 2))
    count = sums[0]
    sum_conf = sums[1]
    sum_acc = sums[2]
    count = count.at[14].add(count[15])[:15]
    sum_conf = sum_conf.at[14].add(sum_conf[15])[:15]
    sum_acc = sum_acc.at[14].add(sum_acc[15])[:15]
    total = jnp.float32(n)
    denom = jnp.maximum(count, 1.0)
    contrib = jnp.where(
        count > 0.0,
        jnp.abs(sum_conf / denom - sum_acc / denom) * (count / total),
        0.0,
    )
    return jnp.sum(contrib, keepdims=True)


def kernel(logits, mask, targets):
    return _sc_kernel(logits, mask, targets)


def _tc_kernel(logits, mask, targets):
    part = _partial_sums(logits, mask, targets)
    cum = part.reshape(8, 3)
    zero = jnp.zeros((1, 3), jnp.float32)
    per_bin = cum - jnp.concatenate([cum[1:], zero], axis=0)
    count = per_bin[:, 0]
    sum_conf = per_bin[:, 1]
    sum_acc = per_bin[:, 2]
    total = jnp.float32(logits.size)
    denom = jnp.maximum(count, 1.0)
    contrib = jnp.where(
        count > 0.0,
        jnp.abs(sum_conf / denom - sum_acc / denom) * (count / total),
        0.0,
    )
    return jnp.sum(contrib, keepdims=True)


# speedup vs baseline: 3.6421x; 3.6421x over previous
import functools

import jax
import jax.numpy as jnp
from jax import lax
from jax.experimental import pallas as pl
from jax.experimental.pallas import tpu as pltpu
from jax.experimental.pallas import tpu_sc as plsc

_THRESH = (0.5333333611488342, 0.6000000238418579, 0.6666666865348816,
           0.7333333492279053, 0.8000000715255737, 0.8666667342185974,
           0.9333333969116211)

_ROWS = 8192
_COLS = 2048
_BLOCK_ROWS = 256
_GRID = _ROWS // _BLOCK_ROWS


def _ece_body(x_ref, m_ref, t_ref, out_ref):
    @pl.when(pl.program_id(0) == 0)
    def _init():
        for k in range(24):
            out_ref[k] = 0.0

    x = x_ref[...]
    mf = m_ref[...].astype(jnp.float32)
    t = t_ref[...]
    conf = 0.5 * jnp.tanh(0.5 * jnp.abs(x)) + 0.5
    acc = jnp.where(x > 0, t, 1.0 - t) * mf
    confm = conf * mf
    out_ref[0] += jnp.sum(mf)
    out_ref[1] += jnp.sum(confm)
    out_ref[2] += jnp.sum(acc)
    for k, th in enumerate(_THRESH):
        g = conf > th
        base = 3 * (k + 1)
        out_ref[base + 0] += jnp.sum(jnp.where(g, mf, 0.0))
        out_ref[base + 1] += jnp.sum(jnp.where(g, confm, 0.0))
        out_ref[base + 2] += jnp.sum(jnp.where(g, acc, 0.0))


def _partial_sums(logits, mask, targets, interpret=False):
    blk = pl.BlockSpec((_BLOCK_ROWS, _COLS), lambda i: (i, 0))
    return pl.pallas_call(
        _ece_body,
        grid=(_GRID,),
        in_specs=[blk, blk, blk],
        out_specs=pl.BlockSpec(memory_space=pltpu.SMEM),
        out_shape=jax.ShapeDtypeStruct((24,), jnp.float32),
        interpret=interpret,
    )(logits, mask, targets)


_NW = 32
_ELEMS = _ROWS * _COLS
_PER_TILE = _ELEMS // _NW
_CHUNK = 16384
_NPIECE = _PER_TILE // _CHUNK
_NVEC = _CHUNK // 16
_UNROLL = 8


def _sc_hist_call(logits_flat, maskw, targets_flat):
    mesh = plsc.VectorSubcoreMesh(core_axis_name="c", subcore_axis_name="s")

    @functools.partial(
        pl.kernel,
        out_type=jax.ShapeDtypeStruct((_NW, 768), jnp.float32),
        mesh=mesh,
        compiler_params=pltpu.CompilerParams(needs_layout_passes=False),
        scratch_types=[
            pltpu.VMEM((2 * _CHUNK,), jnp.float32),
            pltpu.VMEM((2 * _CHUNK,), jnp.float32),
            pltpu.VMEM((2 * _CHUNK,), jnp.float32),
        ] + [pltpu.VMEM((768,), jnp.float32) for _ in range(_UNROLL)] + [
            pltpu.SemaphoreType.DMA((2,)),
            pltpu.SemaphoreType.DMA((2,)),
            pltpu.SemaphoreType.DMA((2,)),
        ],
    )
    def sc_ece(x_hbm, mw_hbm, t_hbm, out_hbm, xbuf, tbuf, mbuf, *rest):
        hists = rest[:_UNROLL]
        xsem, tsem, msem = rest[_UNROLL:]
        wid = lax.axis_index("s") * 2 + lax.axis_index("c")
        base = wid * _PER_TILE

        zeros16 = jnp.zeros((16,), jnp.float32)
        for h in hists:
            for k in range(48):
                h[pl.ds(16 * k, 16)] = zeros16

        iota = lax.iota(jnp.int32, 16)
        lane_off = iota * 16
        ones16 = jnp.ones((16,), jnp.float32)

        def slot_refs(p):
            slot = p & 1
            sbase = pl.multiple_of(slot * _CHUNK, _CHUNK)
            return (xbuf.at[pl.ds(sbase, _CHUNK)],
                    tbuf.at[pl.ds(sbase, _CHUNK)],
                    mbuf.at[pl.ds(sbase, _CHUNK)], slot)

        def start_piece(p):
            xb, tb, mb, slot = slot_refs(p)
            pbase = pl.multiple_of(base + p * _CHUNK, _CHUNK)
            pltpu.async_copy(x_hbm.at[pl.ds(pbase, _CHUNK)], xb, xsem.at[slot])
            pltpu.async_copy(t_hbm.at[pl.ds(pbase, _CHUNK)], tb, tsem.at[slot])
            pltpu.async_copy(mw_hbm.at[pl.ds(pbase, _CHUNK)], mb,
                             msem.at[slot])

        def wait_piece(p):
            xb, tb, mb, slot = slot_refs(p)
            pltpu.make_async_copy(x_hbm.at[pl.ds(0, _CHUNK)], xb,
                                  xsem.at[slot]).wait()
            pltpu.make_async_copy(t_hbm.at[pl.ds(0, _CHUNK)], tb,
                                  tsem.at[slot]).wait()
            pltpu.make_async_copy(mw_hbm.at[pl.ds(0, _CHUNK)], mb,
                                  msem.at[slot]).wait()

        start_piece(0)

        def piece_body(p, _):
            wait_piece(p)

            @pl.when(p + 1 < _NPIECE)
            def _prefetch():
                start_piece(p + 1)

            xb, tb, mbf, slot = slot_refs(p)

            def vec_body(v, _):
                for u in range(_UNROLL):
                    i = _UNROLL * v + u
                    off = pl.multiple_of(16 * i, 16)
                    x = xb[pl.ds(off, 16)]
                    t = tb[pl.ds(off, 16)]
                    mb = mbf[pl.ds(off, 16)] != 0.0
                    conf = 1.0 / (1.0 + jnp.exp(-jnp.abs(x)))
                    acc = jnp.where(x > 0.0, t, 1.0 - t)
                    b = (conf * 15.0).astype(jnp.int32)
                    idx = lane_off + b
                    h = hists[u]
                    plsc.addupdate_scatter(h, [idx], ones16, mask=mb)
                    plsc.addupdate_scatter(h, [idx + 256], conf, mask=mb)
                    plsc.addupdate_scatter(h, [idx + 512], acc, mask=mb)
                return 0

            lax.fori_loop(0, _NVEC // _UNROLL, vec_body, 0)
            return 0

        lax.fori_loop(0, _NPIECE, piece_body, 0)
        for k in range(48):
            sl = pl.ds(16 * k, 16)
            tot = hists[0][sl]
            for h in hists[1:]:
                tot = tot + h[sl]
            hists[0][sl] = tot
        pltpu.sync_copy(hists[0], out_hbm.at[wid])

    return sc_ece(logits_flat, maskw, targets_flat)


def _sc_kernel(logits, mask, targets):
    n = logits.size
    mask_f = mask.astype(jnp.float32).reshape(n)
    part = _sc_hist_call(logits.reshape(n), mask_f, targets.reshape(n))
    sums = part.reshape(_NW, 3, 16, 16).sum(axis=(0, 2))
    count = sums[0]
    sum_conf = sums[1]
    sum_acc = sums[2]
    count = count.at[14].add(count[15])[:15]
    sum_conf = sum_conf.at[14].add(sum_conf[15])[:15]
    sum_acc = sum_acc.at[14].add(sum_acc[15])[:15]
    total = jnp.float32(n)
    denom = jnp.maximum(count, 1.0)
    contrib = jnp.where(
        count > 0.0,
        jnp.abs(sum_conf / denom - sum_acc / denom) * (count / total),
        0.0,
    )
    return jnp.sum(contrib, keepdims=True)


def kernel(logits, mask, targets):
    return _sc_kernel(logits, mask, targets)


def _tc_kernel(logits, mask, targets):
    part = _partial_sums(logits, mask, targets)
    cum = part.reshape(8, 3)
    zero = jnp.zeros((1, 3), jnp.float32)
    per_bin = cum - jnp.concatenate([cum[1:], zero], axis=0)
    count = per_bin[:, 0]
    sum_conf = per_bin[:, 1]
    sum_acc = per_bin[:, 2]
    total = jnp.float32(logits.size)
    denom = jnp.maximum(count, 1.0)
    contrib = jnp.where(
        count > 0.0,
        jnp.abs(sum_conf / denom - sum_acc / denom) * (count / total),
        0.0,
    )
    return jnp.sum(contrib, keepdims=True)


# speedup vs baseline: 9.2553x; 2.5412x over previous
import functools

import jax
import jax.numpy as jnp
from jax import lax
from jax.experimental import pallas as pl
from jax.experimental.pallas import tpu as pltpu
from jax.experimental.pallas import tpu_sc as plsc

_THRESH = (0.5333333611488342, 0.6000000238418579, 0.6666666865348816,
           0.7333333492279053, 0.8000000715255737, 0.8666667342185974,
           0.9333333969116211)

_ROWS = 8192
_COLS = 2048
_BLOCK_ROWS = 256
_GRID = _ROWS // _BLOCK_ROWS


def _ece_body(x_ref, m_ref, t_ref, out_ref):
    @pl.when(pl.program_id(0) == 0)
    def _init():
        for k in range(24):
            out_ref[k] = 0.0

    x = x_ref[...]
    mf = m_ref[...].astype(jnp.float32)
    t = t_ref[...]
    conf = 0.5 * jnp.tanh(0.5 * jnp.abs(x)) + 0.5
    acc = jnp.where(x > 0, t, 1.0 - t) * mf
    confm = conf * mf
    out_ref[0] += jnp.sum(mf)
    out_ref[1] += jnp.sum(confm)
    out_ref[2] += jnp.sum(acc)
    for k, th in enumerate(_THRESH):
        g = conf > th
        base = 3 * (k + 1)
        out_ref[base + 0] += jnp.sum(jnp.where(g, mf, 0.0))
        out_ref[base + 1] += jnp.sum(jnp.where(g, confm, 0.0))
        out_ref[base + 2] += jnp.sum(jnp.where(g, acc, 0.0))


def _partial_sums(logits, mask, targets, interpret=False):
    blk = pl.BlockSpec((_BLOCK_ROWS, _COLS), lambda i: (i, 0))
    return pl.pallas_call(
        _ece_body,
        grid=(_GRID,),
        in_specs=[blk, blk, blk],
        out_specs=pl.BlockSpec(memory_space=pltpu.SMEM),
        out_shape=jax.ShapeDtypeStruct((24,), jnp.float32),
        interpret=interpret,
    )(logits, mask, targets)


_NW = 32
_ELEMS = _ROWS * _COLS
_PER_TILE = _ELEMS // _NW
_CHUNK = 16384
_NPIECE = _PER_TILE // _CHUNK
_NVEC = _CHUNK // 16
_UNROLL = 8


def _sc_hist_call(logits_flat, maskw, targets_flat):
    mesh = plsc.VectorSubcoreMesh(core_axis_name="c", subcore_axis_name="s")

    @functools.partial(
        pl.kernel,
        out_type=jax.ShapeDtypeStruct((_NW, 768), jnp.float32),
        mesh=mesh,
        compiler_params=pltpu.CompilerParams(needs_layout_passes=False),
        scratch_types=[
            pltpu.VMEM((2 * _CHUNK,), jnp.float32),
            pltpu.VMEM((2 * _CHUNK,), jnp.float32),
            pltpu.VMEM((2 * _CHUNK,), jnp.float32),
        ] + [pltpu.VMEM((768,), jnp.float32) for _ in range(_UNROLL)] + [
            pltpu.SemaphoreType.DMA((2,)),
            pltpu.SemaphoreType.DMA((2,)),
            pltpu.SemaphoreType.DMA((2,)),
        ],
    )
    def sc_ece(x_hbm, mw_hbm, t_hbm, out_hbm, xbuf, tbuf, mbuf, *rest):
        hists = rest[:_UNROLL]
        xsem, tsem, msem = rest[_UNROLL:]
        wid = lax.axis_index("s") * 2 + lax.axis_index("c")
        base = wid * _PER_TILE

        zeros16 = jnp.zeros((16,), jnp.float32)
        for h in hists:
            for k in range(48):
                h[pl.ds(16 * k, 16)] = zeros16

        iota = lax.iota(jnp.int32, 16)
        lane_off = iota * 16
        ones16 = jnp.ones((16,), jnp.float32)
        hcnt = [h.at[pl.ds(0, 256)] for h in hists]
        hcnf = [h.at[pl.ds(256, 256)] for h in hists]
        hacc = [h.at[pl.ds(512, 256)] for h in hists]

        def slot_refs(p):
            slot = p & 1
            sbase = pl.multiple_of(slot * _CHUNK, _CHUNK)
            return (xbuf.at[pl.ds(sbase, _CHUNK)],
                    tbuf.at[pl.ds(sbase, _CHUNK)],
                    mbuf.at[pl.ds(sbase, _CHUNK)], slot)

        def start_piece(p):
            xb, tb, mb, slot = slot_refs(p)
            pbase = pl.multiple_of(base + p * _CHUNK, _CHUNK)
            pltpu.async_copy(x_hbm.at[pl.ds(pbase, _CHUNK)], xb, xsem.at[slot])
            pltpu.async_copy(t_hbm.at[pl.ds(pbase, _CHUNK)], tb, tsem.at[slot])
            pltpu.async_copy(mw_hbm.at[pl.ds(pbase, _CHUNK)], mb,
                             msem.at[slot])

        def wait_piece(p):
            xb, tb, mb, slot = slot_refs(p)
            pltpu.make_async_copy(x_hbm.at[pl.ds(0, _CHUNK)], xb,
                                  xsem.at[slot]).wait()
            pltpu.make_async_copy(t_hbm.at[pl.ds(0, _CHUNK)], tb,
                                  tsem.at[slot]).wait()
            pltpu.make_async_copy(mw_hbm.at[pl.ds(0, _CHUNK)], mb,
                                  msem.at[slot]).wait()

        start_piece(0)

        def piece_body(p, _):
            wait_piece(p)

            @pl.when(p + 1 < _NPIECE)
            def _prefetch():
                start_piece(p + 1)

            xb, tb, mbf, slot = slot_refs(p)

            def vec_body(v, _):
                offs = [pl.multiple_of(16 * (_UNROLL * v + u), 16)
                        for u in range(_UNROLL)]
                xs = [xb[pl.ds(o, 16)] for o in offs]
                ts = [tb[pl.ds(o, 16)] for o in offs]
                mbs = [mbf[pl.ds(o, 16)] > 0.5 for o in offs]
                es = [jnp.exp(-jnp.abs(x)) for x in xs]
                confs = [1.0 / (1.0 + e) for e in es]
                accs = [jnp.where(x > 0.0, t, 1.0 - t)
                        for x, t in zip(xs, ts)]
                idxs = [lane_off + (c * 15.0).astype(jnp.int32)
                        for c in confs]
                for u in range(_UNROLL):
                    plsc.addupdate_scatter(hcnt[u], [idxs[u]], ones16,
                                           mask=mbs[u])
                for u in range(_UNROLL):
                    plsc.addupdate_scatter(hcnf[u], [idxs[u]], confs[u],
                                           mask=mbs[u])
                for u in range(_UNROLL):
                    plsc.addupdate_scatter(hacc[u], [idxs[u]], accs[u],
                                           mask=mbs[u])
                return 0

            lax.fori_loop(0, _NVEC // _UNROLL, vec_body, 0)
            return 0

        lax.fori_loop(0, _NPIECE, piece_body, 0)
        for k in range(48):
            sl = pl.ds(16 * k, 16)
            tot = hists[0][sl]
            for h in hists[1:]:
                tot = tot + h[sl]
            hists[0][sl] = tot
        pltpu.sync_copy(hists[0], out_hbm.at[wid])

    return sc_ece(logits_flat, maskw, targets_flat)


def _sc_kernel(logits, mask, targets):
    n = logits.size
    mask_f = mask.astype(jnp.float32).reshape(n)
    part = _sc_hist_call(logits.reshape(n), mask_f, targets.reshape(n))
    sums = part.reshape(_NW, 3, 16, 16).sum(axis=(0, 2))
    count = sums[0]
    sum_conf = sums[1]
    sum_acc = sums[2]
    count = count.at[14].add(count[15])[:15]
    sum_conf = sum_conf.at[14].add(sum_conf[15])[:15]
    sum_acc = sum_acc.at[14].add(sum_acc[15])[:15]
    total = jnp.float32(n)
    denom = jnp.maximum(count, 1.0)
    contrib = jnp.where(
        count > 0.0,
        jnp.abs(sum_conf / denom - sum_acc / denom) * (count / total),
        0.0,
    )
    return jnp.sum(contrib, keepdims=True)


def kernel(logits, mask, targets):
    return _sc_kernel(logits, mask, targets)


def _tc_kernel(logits, mask, targets):
    part = _partial_sums(logits, mask, targets)
    cum = part.reshape(8, 3)
    zero = jnp.zeros((1, 3), jnp.float32)
    per_bin = cum - jnp.concatenate([cum[1:], zero], axis=0)
    count = per_bin[:, 0]
    sum_conf = per_bin[:, 1]
    sum_acc = per_bin[:, 2]
    total = jnp.float32(logits.size)
    denom = jnp.maximum(count, 1.0)
    contrib = jnp.where(
        count > 0.0,
        jnp.abs(sum_conf / denom - sum_acc / denom) * (count / total),
        0.0,
    )
    return jnp.sum(contrib, keepdims=True)


# speedup vs baseline: 13.7556x; 1.4862x over previous
import functools

import jax
import jax.numpy as jnp
from jax import lax
from jax.experimental import pallas as pl
from jax.experimental.pallas import tpu as pltpu
from jax.experimental.pallas import tpu_sc as plsc

_THRESH = (0.5333333611488342, 0.6000000238418579, 0.6666666865348816,
           0.7333333492279053, 0.8000000715255737, 0.8666667342185974,
           0.9333333969116211)

_ROWS = 8192
_COLS = 2048
_BLOCK_ROWS = 256
_GRID = _ROWS // _BLOCK_ROWS


def _ece_body(x_ref, m_ref, t_ref, out_ref):
    @pl.when(pl.program_id(0) == 0)
    def _init():
        for k in range(24):
            out_ref[k] = 0.0

    x = x_ref[...]
    mf = m_ref[...].astype(jnp.float32)
    t = t_ref[...]
    conf = 0.5 * jnp.tanh(0.5 * jnp.abs(x)) + 0.5
    acc = jnp.where(x > 0, t, 1.0 - t) * mf
    confm = conf * mf
    out_ref[0] += jnp.sum(mf)
    out_ref[1] += jnp.sum(confm)
    out_ref[2] += jnp.sum(acc)
    for k, th in enumerate(_THRESH):
        g = conf > th
        base = 3 * (k + 1)
        out_ref[base + 0] += jnp.sum(jnp.where(g, mf, 0.0))
        out_ref[base + 1] += jnp.sum(jnp.where(g, confm, 0.0))
        out_ref[base + 2] += jnp.sum(jnp.where(g, acc, 0.0))


def _partial_sums(logits, mask, targets, interpret=False):
    blk = pl.BlockSpec((_BLOCK_ROWS, _COLS), lambda i: (i, 0))
    return pl.pallas_call(
        _ece_body,
        grid=(_GRID,),
        in_specs=[blk, blk, blk],
        out_specs=pl.BlockSpec(memory_space=pltpu.SMEM),
        out_shape=jax.ShapeDtypeStruct((24,), jnp.float32),
        interpret=interpret,
    )(logits, mask, targets)


_NW = 32
_ELEMS = _ROWS * _COLS
_PER_TILE = _ELEMS // _NW
_CHUNK = 16384
_NPIECE = _PER_TILE // _CHUNK
_NVEC = _CHUNK // 16
_UNROLL = 8


_ROWS_PER_TILE = _ROWS // _NW
_PIECE_ROWS = 8
_NPIECE2 = _ROWS_PER_TILE // _PIECE_ROWS
_VPR = _COLS // 16


def _sc_hist_call(logits2d, maskf2d, targets2d):
    mesh = plsc.VectorSubcoreMesh(core_axis_name="c", subcore_axis_name="s")

    @functools.partial(
        pl.kernel,
        out_type=jax.ShapeDtypeStruct((_NW, 768), jnp.float32),
        mesh=mesh,
        compiler_params=pltpu.CompilerParams(
            needs_layout_passes=False, use_tc_tiling_on_sc=True),
        scratch_types=[
            pltpu.VMEM((2 * _PIECE_ROWS, _COLS), jnp.float32),
            pltpu.VMEM((2 * _PIECE_ROWS, _COLS), jnp.float32),
            pltpu.VMEM((2 * _PIECE_ROWS, _COLS), jnp.float32),
        ] + [pltpu.VMEM((768,), jnp.float32) for _ in range(_UNROLL)] + [
            pltpu.SemaphoreType.DMA((2,)),
            pltpu.SemaphoreType.DMA((2,)),
            pltpu.SemaphoreType.DMA((2,)),
        ],
    )
    def sc_ece(x_hbm, mw_hbm, t_hbm, out_hbm, xbuf, tbuf, mbuf, *rest):
        hists = rest[:_UNROLL]
        xsem, tsem, msem = rest[_UNROLL:]
        wid = lax.axis_index("s") * 2 + lax.axis_index("c")
        row_base = wid * _ROWS_PER_TILE

        zeros16 = jnp.zeros((16,), jnp.float32)
        for h in hists:
            for k in range(48):
                h[pl.ds(16 * k, 16)] = zeros16

        iota = lax.iota(jnp.int32, 16)
        lane_off = iota * 16
        ones16 = jnp.ones((16,), jnp.float32)
        hcnt = [h.at[pl.ds(0, 256)] for h in hists]
        hcnf = [h.at[pl.ds(256, 256)] for h in hists]
        hacc = [h.at[pl.ds(512, 256)] for h in hists]

        def start_piece(p, slot):
            r0 = pl.multiple_of(row_base + p * _PIECE_ROWS, _PIECE_ROWS)
            dst = pl.ds(slot * _PIECE_ROWS, _PIECE_ROWS)
            pltpu.async_copy(x_hbm.at[pl.ds(r0, _PIECE_ROWS), :],
                             xbuf.at[dst, :], xsem.at[slot])
            pltpu.async_copy(t_hbm.at[pl.ds(r0, _PIECE_ROWS), :],
                             tbuf.at[dst, :], tsem.at[slot])
            pltpu.async_copy(mw_hbm.at[pl.ds(r0, _PIECE_ROWS), :],
                             mbuf.at[dst, :], msem.at[slot])

        def wait_piece(slot):
            src = pl.ds(0, _PIECE_ROWS)
            dst = pl.ds(slot * _PIECE_ROWS, _PIECE_ROWS)
            pltpu.make_async_copy(x_hbm.at[src, :], xbuf.at[dst, :],
                                  xsem.at[slot]).wait()
            pltpu.make_async_copy(t_hbm.at[src, :], tbuf.at[dst, :],
                                  tsem.at[slot]).wait()
            pltpu.make_async_copy(mw_hbm.at[src, :], mbuf.at[dst, :],
                                  msem.at[slot]).wait()

        def compute_piece(slot):
            for r in range(_PIECE_ROWS):
                row = slot * _PIECE_ROWS + r

                def vec_body(v, _):
                    offs = [pl.multiple_of(16 * (_UNROLL * v + u), 16)
                            for u in range(_UNROLL)]
                    xs = [xbuf[row, pl.ds(o, 16)] for o in offs]
                    ts = [tbuf[row, pl.ds(o, 16)] for o in offs]
                    mbs = [mbuf[row, pl.ds(o, 16)] > 0.5 for o in offs]
                    es = [jnp.exp(-jnp.abs(x)) for x in xs]
                    confs = [1.0 / (1.0 + e) for e in es]
                    accs = [jnp.where(x > 0.0, t, 1.0 - t)
                            for x, t in zip(xs, ts)]
                    idxs = [lane_off + (c * 15.0).astype(jnp.int32)
                            for c in confs]
                    for u in range(_UNROLL):
                        plsc.addupdate_scatter(hcnt[u], [idxs[u]], ones16,
                                               mask=mbs[u])
                    for u in range(_UNROLL):
                        plsc.addupdate_scatter(hcnf[u], [idxs[u]], confs[u],
                                               mask=mbs[u])
                    for u in range(_UNROLL):
                        plsc.addupdate_scatter(hacc[u], [idxs[u]], accs[u],
                                               mask=mbs[u])
                    return 0

                lax.fori_loop(0, _VPR // _UNROLL, vec_body, 0)

        start_piece(0, 0)
        start_piece(1, 1)

        def pair_body(s, _):
            for slot in range(2):
                p = 2 * s + slot
                wait_piece(slot)
                compute_piece(slot)

                @pl.when(p + 2 < _NPIECE2)
                def _prefetch():
                    start_piece(p + 2, slot)
            return 0

        lax.fori_loop(0, _NPIECE2 // 2, pair_body, 0)
        for k in range(48):
            sl = pl.ds(16 * k, 16)
            tot = hists[0][sl]
            for h in hists[1:]:
                tot = tot + h[sl]
            hists[0][sl] = tot
        pltpu.sync_copy(hists[0], out_hbm.at[wid])

    return sc_ece(logits2d, maskf2d, targets2d)


def _sc_kernel(logits, mask, targets):
    n = logits.size
    mask_f = mask.astype(jnp.float32)
    part = _sc_hist_call(logits, mask_f, targets)
    sums = part.reshape(_NW, 3, 16, 16).sum(axis=(0, 2))
    count = sums[0]
    sum_conf = sums[1]
    sum_acc = sums[2]
    count = count.at[14].add(count[15])[:15]
    sum_conf = sum_conf.at[14].add(sum_conf[15])[:15]
    sum_acc = sum_acc.at[14].add(sum_acc[15])[:15]
    total = jnp.float32(n)
    denom = jnp.maximum(count, 1.0)
    contrib = jnp.where(
        count > 0.0,
        jnp.abs(sum_conf / denom - sum_acc / denom) * (count / total),
        0.0,
    )
    return jnp.sum(contrib, keepdims=True)


def kernel(logits, mask, targets):
    return _sc_kernel(logits, mask, targets)


def _tc_kernel(logits, mask, targets):
    part = _partial_sums(logits, mask, targets)
    cum = part.reshape(8, 3)
    zero = jnp.zeros((1, 3), jnp.float32)
    per_bin = cum - jnp.concatenate([cum[1:], zero], axis=0)
    count = per_bin[:, 0]
    sum_conf = per_bin[:, 1]
    sum_acc = per_bin[:, 2]
    total = jnp.float32(logits.size)
    denom = jnp.maximum(count, 1.0)
    contrib = jnp.where(
        count > 0.0,
        jnp.abs(sum_conf / denom - sum_acc / denom) * (count / total),
        0.0,
    )
    return jnp.sum(contrib, keepdims=True)


# speedup vs baseline: 15.7269x; 1.1433x over previous
import functools

import jax
import jax.numpy as jnp
from jax import lax
from jax.experimental import pallas as pl
from jax.experimental.pallas import tpu as pltpu
from jax.experimental.pallas import tpu_sc as plsc

_THRESH = (0.5333333611488342, 0.6000000238418579, 0.6666666865348816,
           0.7333333492279053, 0.8000000715255737, 0.8666667342185974,
           0.9333333969116211)

_ROWS = 8192
_COLS = 2048
_BLOCK_ROWS = 256
_GRID = _ROWS // _BLOCK_ROWS


def _ece_body(x_ref, m_ref, t_ref, out_ref):
    @pl.when(pl.program_id(0) == 0)
    def _init():
        for k in range(24):
            out_ref[k] = 0.0

    x = x_ref[...]
    mf = m_ref[...].astype(jnp.float32)
    t = t_ref[...]
    conf = 0.5 * jnp.tanh(0.5 * jnp.abs(x)) + 0.5
    acc = jnp.where(x > 0, t, 1.0 - t) * mf
    confm = conf * mf
    out_ref[0] += jnp.sum(mf)
    out_ref[1] += jnp.sum(confm)
    out_ref[2] += jnp.sum(acc)
    for k, th in enumerate(_THRESH):
        g = conf > th
        base = 3 * (k + 1)
        out_ref[base + 0] += jnp.sum(jnp.where(g, mf, 0.0))
        out_ref[base + 1] += jnp.sum(jnp.where(g, confm, 0.0))
        out_ref[base + 2] += jnp.sum(jnp.where(g, acc, 0.0))


def _partial_sums(logits, mask, targets, interpret=False):
    blk = pl.BlockSpec((_BLOCK_ROWS, _COLS), lambda i: (i, 0))
    return pl.pallas_call(
        _ece_body,
        grid=(_GRID,),
        in_specs=[blk, blk, blk],
        out_specs=pl.BlockSpec(memory_space=pltpu.SMEM),
        out_shape=jax.ShapeDtypeStruct((24,), jnp.float32),
        interpret=interpret,
    )(logits, mask, targets)


_NW = 32
_ELEMS = _ROWS * _COLS
_PER_TILE = _ELEMS // _NW
_CHUNK = 16384
_NPIECE = _PER_TILE // _CHUNK
_NVEC = _CHUNK // 16
_UNROLL = 8


_ROWS_PER_TILE = _ROWS // _NW
_PIECE_ROWS = 8
_NPIECE2 = _ROWS_PER_TILE // _PIECE_ROWS
_VPR = _COLS // 16


def _sc_hist_call(logits2d, maskf2d, targets2d):
    mesh = plsc.VectorSubcoreMesh(core_axis_name="c", subcore_axis_name="s")

    @functools.partial(
        pl.kernel,
        out_type=jax.ShapeDtypeStruct((_NW, _UNROLL, 512), jnp.float32),
        mesh=mesh,
        compiler_params=pltpu.CompilerParams(
            needs_layout_passes=False, use_tc_tiling_on_sc=True),
        scratch_types=[
            pltpu.VMEM((2 * _PIECE_ROWS, _COLS), jnp.float32),
            pltpu.VMEM((2 * _PIECE_ROWS, _COLS), jnp.float32),
            pltpu.VMEM((2 * _PIECE_ROWS, _COLS), jnp.float32),
        ] + [pltpu.VMEM((512,), jnp.float32) for _ in range(_UNROLL)] + [
            pltpu.SemaphoreType.DMA((2,)),
            pltpu.SemaphoreType.DMA((2,)),
            pltpu.SemaphoreType.DMA((2,)),
        ],
    )
    def sc_ece(x_hbm, mw_hbm, t_hbm, out_hbm, xbuf, tbuf, mbuf, *rest):
        hists = rest[:_UNROLL]
        xsem, tsem, msem = rest[_UNROLL:]
        wid = lax.axis_index("s") * 2 + lax.axis_index("c")
        row_base = wid * _ROWS_PER_TILE

        zeros16 = jnp.zeros((16,), jnp.float32)
        for h in hists:
            for k in range(32):
                h[pl.ds(16 * k, 16)] = zeros16

        iota = lax.iota(jnp.int32, 16)
        lane_off = iota * 16
        hpak = [h.at[pl.ds(0, 256)] for h in hists]
        hcnf = [h.at[pl.ds(256, 256)] for h in hists]

        def start_piece(p, slot):
            r0 = pl.multiple_of(row_base + p * _PIECE_ROWS, _PIECE_ROWS)
            dst = pl.ds(slot * _PIECE_ROWS, _PIECE_ROWS)
            pltpu.async_copy(x_hbm.at[pl.ds(r0, _PIECE_ROWS), :],
                             xbuf.at[dst, :], xsem.at[slot])
            pltpu.async_copy(t_hbm.at[pl.ds(r0, _PIECE_ROWS), :],
                             tbuf.at[dst, :], tsem.at[slot])
            pltpu.async_copy(mw_hbm.at[pl.ds(r0, _PIECE_ROWS), :],
                             mbuf.at[dst, :], msem.at[slot])

        def wait_piece(slot):
            src = pl.ds(0, _PIECE_ROWS)
            dst = pl.ds(slot * _PIECE_ROWS, _PIECE_ROWS)
            pltpu.make_async_copy(x_hbm.at[src, :], xbuf.at[dst, :],
                                  xsem.at[slot]).wait()
            pltpu.make_async_copy(t_hbm.at[src, :], tbuf.at[dst, :],
                                  tsem.at[slot]).wait()
            pltpu.make_async_copy(mw_hbm.at[src, :], mbuf.at[dst, :],
                                  msem.at[slot]).wait()

        def compute_piece(slot):
            for r in range(_PIECE_ROWS):
                row = slot * _PIECE_ROWS + r

                def vec_body(v, _):
                    offs = [pl.multiple_of(16 * (_UNROLL * v + u), 16)
                            for u in range(_UNROLL)]
                    xs = [xbuf[row, pl.ds(o, 16)] for o in offs]
                    ts = [tbuf[row, pl.ds(o, 16)] for o in offs]
                    mbs = [mbuf[row, pl.ds(o, 16)] > 0.5 for o in offs]
                    es = [jnp.exp(-jnp.abs(x)) for x in xs]
                    confs = [1.0 / (1.0 + e) for e in es]
                    tcs = [t * 0.000244140625 for t in ts]
                    paks = [jnp.where(x > 0.0, 1.0 + tc,
                                      1.000244140625 - tc)
                            for x, tc in zip(xs, tcs)]
                    idxs = [lane_off + (c * 15.0).astype(jnp.int32)
                            for c in confs]
                    for u in range(_UNROLL):
                        plsc.addupdate_scatter(hpak[u], [idxs[u]], paks[u],
                                               mask=mbs[u])
                    for u in range(_UNROLL):
                        plsc.addupdate_scatter(hcnf[u], [idxs[u]], confs[u],
                                               mask=mbs[u])
                    return 0

                lax.fori_loop(0, _VPR // _UNROLL, vec_body, 0)

        start_piece(0, 0)
        start_piece(1, 1)

        def pair_body(s, _):
            for slot in range(2):
                p = 2 * s + slot
                wait_piece(slot)
                compute_piece(slot)

                @pl.when(p + 2 < _NPIECE2)
                def _prefetch():
                    start_piece(p + 2, slot)
            return 0

        lax.fori_loop(0, _NPIECE2 // 2, pair_body, 0)
        for u in range(_UNROLL):
            pltpu.sync_copy(hists[u], out_hbm.at[wid, u])

    return sc_ece(logits2d, maskf2d, targets2d)


def _sc_kernel(logits, mask, targets):
    n = logits.size
    mask_f = mask.astype(jnp.float32)
    part = _sc_hist_call(logits, mask_f, targets)
    q = part.reshape(_NW, _UNROLL, 2, 16, 16)
    pak = q[:, :, 0]
    count_g = jnp.floor(pak)
    acc_g = (pak - count_g) * 4096.0
    count = count_g.sum(axis=(0, 1, 2))
    sum_acc = acc_g.sum(axis=(0, 1, 2))
    sum_conf = q[:, :, 1].sum(axis=(0, 1, 2))
    count = count.at[14].add(count[15])[:15]
    sum_conf = sum_conf.at[14].add(sum_conf[15])[:15]
    sum_acc = sum_acc.at[14].add(sum_acc[15])[:15]
    total = jnp.float32(n)
    denom = jnp.maximum(count, 1.0)
    contrib = jnp.where(
        count > 0.0,
        jnp.abs(sum_conf / denom - sum_acc / denom) * (count / total),
        0.0,
    )
    return jnp.sum(contrib, keepdims=True)


def kernel(logits, mask, targets):
    return _sc_kernel(logits, mask, targets)


def _tc_kernel(logits, mask, targets):
    part = _partial_sums(logits, mask, targets)
    cum = part.reshape(8, 3)
    zero = jnp.zeros((1, 3), jnp.float32)
    per_bin = cum - jnp.concatenate([cum[1:], zero], axis=0)
    count = per_bin[:, 0]
    sum_conf = per_bin[:, 1]
    sum_acc = per_bin[:, 2]
    total = jnp.float32(logits.size)
    denom = jnp.maximum(count, 1.0)
    contrib = jnp.where(
        count > 0.0,
        jnp.abs(sum_conf / denom - sum_acc / denom) * (count / total),
        0.0,
    )
    return jnp.sum(contrib, keepdims=True)


# speedup vs baseline: 15.7314x; 1.0003x over previous
import functools

import jax
import jax.numpy as jnp
from jax import lax
from jax.experimental import pallas as pl
from jax.experimental.pallas import tpu as pltpu
from jax.experimental.pallas import tpu_sc as plsc

_ROWS = 8192
_COLS = 2048
_NW = 32
_UNROLL = 8
_ROWS_PER_TILE = _ROWS // _NW
_PIECE_ROWS = 8
_NPIECE = _ROWS_PER_TILE // _PIECE_ROWS
_VPR = _COLS // 16


def _sc_hist_call(logits2d, maskf2d, targets2d):
    mesh = plsc.VectorSubcoreMesh(core_axis_name="c", subcore_axis_name="s")

    @functools.partial(
        pl.kernel,
        out_type=jax.ShapeDtypeStruct((_NW, _UNROLL, 512), jnp.float32),
        mesh=mesh,
        compiler_params=pltpu.CompilerParams(
            needs_layout_passes=False, use_tc_tiling_on_sc=True),
        scratch_types=[
            pltpu.VMEM((2 * _PIECE_ROWS, _COLS), jnp.float32),
            pltpu.VMEM((2 * _PIECE_ROWS, _COLS), jnp.float32),
            pltpu.VMEM((2 * _PIECE_ROWS, _COLS), jnp.float32),
        ] + [pltpu.VMEM((512,), jnp.float32) for _ in range(_UNROLL)] + [
            pltpu.SemaphoreType.DMA((2,)),
            pltpu.SemaphoreType.DMA((2,)),
            pltpu.SemaphoreType.DMA((2,)),
        ],
    )
    def sc_ece(x_hbm, m_hbm, t_hbm, out_hbm, xbuf, tbuf, mbuf, *rest):
        hists = rest[:_UNROLL]
        xsem, tsem, msem = rest[_UNROLL:]
        wid = lax.axis_index("s") * 2 + lax.axis_index("c")
        row_base = wid * _ROWS_PER_TILE

        zeros16 = jnp.zeros((16,), jnp.float32)
        for h in hists:
            for k in range(32):
                h[pl.ds(16 * k, 16)] = zeros16

        lane_off = lax.iota(jnp.int32, 16) * 16
        hpak = [h.at[pl.ds(0, 256)] for h in hists]
        hcnf = [h.at[pl.ds(256, 256)] for h in hists]

        def start_piece(p, slot):
            r0 = pl.multiple_of(row_base + p * _PIECE_ROWS, _PIECE_ROWS)
            dst = pl.ds(slot * _PIECE_ROWS, _PIECE_ROWS)
            pltpu.async_copy(x_hbm.at[pl.ds(r0, _PIECE_ROWS), :],
                             xbuf.at[dst, :], xsem.at[slot])
            pltpu.async_copy(t_hbm.at[pl.ds(r0, _PIECE_ROWS), :],
                             tbuf.at[dst, :], tsem.at[slot])
            pltpu.async_copy(m_hbm.at[pl.ds(r0, _PIECE_ROWS), :],
                             mbuf.at[dst, :], msem.at[slot])

        def wait_piece(slot):
            src = pl.ds(0, _PIECE_ROWS)
            dst = pl.ds(slot * _PIECE_ROWS, _PIECE_ROWS)
            pltpu.make_async_copy(x_hbm.at[src, :], xbuf.at[dst, :],
                                  xsem.at[slot]).wait()
            pltpu.make_async_copy(t_hbm.at[src, :], tbuf.at[dst, :],
                                  tsem.at[slot]).wait()
            pltpu.make_async_copy(m_hbm.at[src, :], mbuf.at[dst, :],
                                  msem.at[slot]).wait()

        def compute_piece(slot):
            for r in range(_PIECE_ROWS):
                row = slot * _PIECE_ROWS + r

                def vec_body(v, _):
                    offs = [pl.multiple_of(16 * (_UNROLL * v + u), 16)
                            for u in range(_UNROLL)]
                    xs = [xbuf[row, pl.ds(o, 16)] for o in offs]
                    ts = [tbuf[row, pl.ds(o, 16)] for o in offs]
                    mbs = [mbuf[row, pl.ds(o, 16)] > 0.5 for o in offs]
                    es = [jnp.exp(-jnp.abs(x)) for x in xs]
                    confs = [1.0 / (1.0 + e) for e in es]
                    tcs = [t * 0.000244140625 for t in ts]
                    paks = [jnp.where(x > 0.0, 1.0 + tc,
                                      1.000244140625 - tc)
                            for x, tc in zip(xs, tcs)]
                    idxs = [lane_off + (c * 15.0).astype(jnp.int32)
                            for c in confs]
                    for u in range(_UNROLL):
                        plsc.addupdate_scatter(hpak[u], [idxs[u]], paks[u],
                                               mask=mbs[u])
                    for u in range(_UNROLL):
                        plsc.addupdate_scatter(hcnf[u], [idxs[u]], confs[u],
                                               mask=mbs[u])
                    return 0

                lax.fori_loop(0, _VPR // _UNROLL, vec_body, 0)

        start_piece(0, 0)
        start_piece(1, 1)

        def pair_body(s, _):
            for slot in range(2):
                p = 2 * s + slot
                wait_piece(slot)
                compute_piece(slot)

                @pl.when(p + 2 < _NPIECE)
                def _prefetch():
                    start_piece(p + 2, slot)
            return 0

        lax.fori_loop(0, _NPIECE // 2, pair_body, 0)
        for u in range(_UNROLL):
            pltpu.sync_copy(hists[u], out_hbm.at[wid, u])

    return sc_ece(logits2d, maskf2d, targets2d)


def kernel(logits, mask, targets):
    mask_f = mask.astype(jnp.float32)
    part = _sc_hist_call(logits, mask_f, targets)
    q = part.reshape(_NW, _UNROLL, 2, 16, 16)
    pak = q[:, :, 0]
    count_g = jnp.floor(pak)
    acc_g = (pak - count_g) * 4096.0
    count = count_g.sum(axis=(0, 1, 2))
    sum_acc = acc_g.sum(axis=(0, 1, 2))
    sum_conf = q[:, :, 1].sum(axis=(0, 1, 2))
    count = count.at[14].add(count[15])[:15]
    sum_conf = sum_conf.at[14].add(sum_conf[15])[:15]
    sum_acc = sum_acc.at[14].add(sum_acc[15])[:15]
    total = jnp.float32(logits.size)
    denom = jnp.maximum(count, 1.0)
    contrib = jnp.where(
        count > 0.0,
        jnp.abs(sum_conf / denom - sum_acc / denom) * (count / total),
        0.0,
    )
    return jnp.sum(contrib, keepdims=True)
